# Initial kernel scaffold; baseline (speedup 1.0000x reference)
import jax
import jax.numpy as jnp
from jax.experimental import pallas as pl


def kernel(x, edge_index, batch, W1, b1, W2, b2):
    raise NotImplementedError("write your pallas kernel here")


# speedup vs baseline: 10.2826x; 10.2826x over previous
import functools
import jax
import jax.numpy as jnp
from jax import lax
from jax.experimental import pallas as pl
from jax.experimental.pallas import tpu as pltpu
from jax.experimental.pallas import tpu_sc as plsc

N = 10000
E = 160000
D_IN = 256
Q = 64
G = 64
BN = 1000

NC = 2
NS = 16

A_CHUNK = 40
A_ITERS = E // (NC * NS * A_CHUNK)

C_CHUNK = 80
C_ITERS = E // (NS * C_CHUNK)

W_TILES = 10
W_ROWS = N // W_TILES
WCHUNK = 200
WITERS = W_ROWS // WCHUNK

_mesh = plsc.VectorSubcoreMesh(core_axis_name="c", subcore_axis_name="s")


@functools.partial(
    pl.kernel,
    mesh=_mesh,
    out_type=[
        jax.ShapeDtypeStruct((N,), jnp.float32),
        jax.ShapeDtypeStruct((N,), jnp.float32),
    ],
    scratch_types=[
        pltpu.VMEM((A_ITERS, A_CHUNK), jnp.int32),
        pltpu.VMEM((A_CHUNK,), jnp.float32),
        pltpu.VMEM((N,), jnp.float32),
        pltpu.VMEM_SHARED((N,), jnp.float32),
    ],
)
def _deg_kernel(dstA_hbm, ones_hbm, zeros_hbm, deg0_hbm, deg1_hbm,
                idx_v, ones_v, stage_v, deg_sp):
    c = lax.axis_index("c")
    s = lax.axis_index("s")

    @pl.when(s == 0)
    def _():
        pltpu.sync_copy(zeros_hbm, deg_sp)
    plsc.subcore_barrier()

    w = c * NS + s
    pltpu.sync_copy(dstA_hbm.at[w], idx_v)
    pltpu.sync_copy(ones_hbm, ones_v)

    def step(i, _):
        pltpu.sync_copy(ones_v, deg_sp.at[idx_v.at[i]], add=True)
        return 0
    lax.fori_loop(0, A_ITERS, step, 0)

    plsc.subcore_barrier()

    @pl.when((s == 0) & (c == 0))
    def _():
        pltpu.sync_copy(deg_sp, stage_v)
        pltpu.sync_copy(stage_v, deg0_hbm)

    @pl.when((s == 0) & (c == 1))
    def _():
        pltpu.sync_copy(deg_sp, stage_v)
        pltpu.sync_copy(stage_v, deg1_hbm)


@functools.partial(
    pl.kernel,
    mesh=_mesh,
    out_type=[jax.ShapeDtypeStruct((N, Q), jnp.float32) for _ in range(4)],
    scratch_types=[
        pltpu.VMEM((C_ITERS, C_CHUNK), jnp.int32),
        pltpu.VMEM((C_ITERS, C_CHUNK), jnp.int32),
        pltpu.VMEM((C_CHUNK, Q), jnp.float32),
        pltpu.VMEM((WCHUNK, Q), jnp.float32),
        pltpu.SemaphoreType.DMA,
        pltpu.VMEM_SHARED((N, Q), jnp.float32),
    ],
    compiler_params=pltpu.CompilerParams(use_tc_tiling_on_sc=False),
)
def _agg_kernel(xs0_hbm, xs1_hbm, xs2_hbm, xs3_hbm, srcC_hbm, dstC_hbm,
                zrows_hbm,
                agg0_hbm, agg1_hbm, agg2_hbm, agg3_hbm,
                sidx_v, didx_v, rows_v, stage_v, sem, agg_sp):
    c = lax.axis_index("c")
    s = lax.axis_index("s")

    pltpu.sync_copy(srcC_hbm.at[s], sidx_v)
    pltpu.sync_copy(dstC_hbm.at[s], didx_v)

    def one_pass(xsq_hbm, aggq_hbm):
        @pl.when(s < W_TILES)
        def _():
            def zstep(j, _):
                pltpu.sync_copy(
                    zrows_hbm,
                    agg_sp.at[pl.ds(s * W_ROWS + j * WCHUNK, WCHUNK)])
                return 0
            lax.fori_loop(0, WITERS, zstep, 0)
        plsc.subcore_barrier()

        def step(i, _):
            pltpu.async_copy(xsq_hbm.at[sidx_v.at[i]], rows_v, sem).wait()
            pltpu.sync_copy(rows_v, agg_sp.at[didx_v.at[i]], add=True)
            return 0
        lax.fori_loop(0, C_ITERS, step, 0)
        plsc.subcore_barrier()

        @pl.when(s < W_TILES)
        def _():
            def wstep(j, _):
                r = s * W_ROWS + j * WCHUNK
                pltpu.sync_copy(agg_sp.at[pl.ds(r, WCHUNK)], stage_v)
                pltpu.sync_copy(stage_v, aggq_hbm.at[pl.ds(r, WCHUNK)])
                return 0
            lax.fori_loop(0, WITERS, wstep, 0)
        plsc.subcore_barrier()

    @pl.when(c == 0)
    def _():
        one_pass(xs0_hbm, agg0_hbm)
        one_pass(xs1_hbm, agg1_hbm)

    @pl.when(c == 1)
    def _():
        one_pass(xs2_hbm, agg2_hbm)
        one_pass(xs3_hbm, agg3_hbm)


def _scale_body(deg0_ref, deg1_ref, x_ref,
                xs0_ref, xs1_ref, xs2_ref, xs3_ref, dis_ref):
    deg = deg0_ref[...] + deg1_ref[...] + 1.0
    dis = lax.rsqrt(deg)
    xv = x_ref[...]
    xs0_ref[...] = dis * xv[:, 0 * Q:1 * Q]
    xs1_ref[...] = dis * xv[:, 1 * Q:2 * Q]
    xs2_ref[...] = dis * xv[:, 2 * Q:3 * Q]
    xs3_ref[...] = dis * xv[:, 3 * Q:4 * Q]
    dis_ref[...] = dis


def _scale_call(deg0, deg1, x):
    return pl.pallas_call(
        _scale_body,
        grid=(N // BN,),
        in_specs=[
            pl.BlockSpec((BN, 1), lambda i: (i, 0)),
            pl.BlockSpec((BN, 1), lambda i: (i, 0)),
            pl.BlockSpec((BN, D_IN), lambda i: (i, 0)),
        ],
        out_specs=[pl.BlockSpec((BN, Q), lambda i: (i, 0))] * 4
        + [pl.BlockSpec((BN, 1), lambda i: (i, 0))],
        out_shape=[jax.ShapeDtypeStruct((N, Q), jnp.float32)] * 4
        + [jax.ShapeDtypeStruct((N, 1), jnp.float32)],
    )(deg0, deg1, x)


def _head_body(x_ref, agg0_ref, agg1_ref, agg2_ref, agg3_ref, dis_ref,
               batch_ref, W1_ref, b1_ref, W2_ref, b2_ref, out_ref, acc_ref):
    i = pl.program_id(0)
    dis = dis_ref[...]
    agg = jnp.concatenate(
        [agg0_ref[...], agg1_ref[...], agg2_ref[...], agg3_ref[...]], axis=1)
    z = agg * dis + (dis * dis) * x_ref[...]
    h = jnp.dot(z, W1_ref[...], preferred_element_type=jnp.float32) \
        + b1_ref[...]
    h = jnp.where(h > 0, h, 0.01 * h)
    v = jnp.dot(h, W2_ref[...], preferred_element_type=jnp.float32)
    b = batch_ref[...]
    gids = lax.broadcasted_iota(jnp.int32, (BN, G), 1)
    onehot = (b == gids).astype(jnp.float32)
    vv = jnp.concatenate([v, jnp.ones((BN, 1), jnp.float32)], axis=1)
    contrib = lax.dot_general(onehot, vv, (((0,), (0,)), ((), ())),
                              preferred_element_type=jnp.float32)

    @pl.when(i == 0)
    def _():
        acc_ref[...] = jnp.zeros_like(acc_ref)

    acc_ref[...] += contrib

    @pl.when(i == pl.num_programs(0) - 1)
    def _():
        a = acc_ref[...]
        out_ref[...] = a[:, 0:1] / jnp.maximum(a[:, 1:2], 1.0) + b2_ref[...]


def _head_call(x, aggs, dis, batch2d, W1, b1r, W2, b2r):
    return pl.pallas_call(
        _head_body,
        grid=(N // BN,),
        in_specs=[
            pl.BlockSpec((BN, D_IN), lambda i: (i, 0)),
            pl.BlockSpec((BN, Q), lambda i: (i, 0)),
            pl.BlockSpec((BN, Q), lambda i: (i, 0)),
            pl.BlockSpec((BN, Q), lambda i: (i, 0)),
            pl.BlockSpec((BN, Q), lambda i: (i, 0)),
            pl.BlockSpec((BN, 1), lambda i: (i, 0)),
            pl.BlockSpec((BN, 1), lambda i: (i, 0)),
            pl.BlockSpec((D_IN, D_IN), lambda i: (0, 0)),
            pl.BlockSpec((1, D_IN), lambda i: (0, 0)),
            pl.BlockSpec((D_IN, 1), lambda i: (0, 0)),
            pl.BlockSpec((1, 1), lambda i: (0, 0)),
        ],
        out_specs=pl.BlockSpec((G, 1), lambda i: (0, 0)),
        out_shape=jax.ShapeDtypeStruct((G, 1), jnp.float32),
        scratch_shapes=[pltpu.VMEM((G, 2), jnp.float32)],
    )(x, *aggs, dis, batch2d, W1, b1r, W2, b2r)


def kernel(x, edge_index, batch, W1, b1, W2, b2):
    ei = edge_index.astype(jnp.int32)
    dstA = ei[1].reshape(NC * NS, A_ITERS, A_CHUNK)
    srcC = ei[0].reshape(NS, C_ITERS, C_CHUNK)
    dstC = ei[1].reshape(NS, C_ITERS, C_CHUNK)
    onesA = jnp.ones((A_CHUNK,), jnp.float32)
    zeros_n = jnp.zeros((N,), jnp.float32)
    zrows = jnp.zeros((WCHUNK, Q), jnp.float32)

    deg0, deg1 = _deg_kernel(dstA, onesA, zeros_n)
    xs0, xs1, xs2, xs3, dis = _scale_call(
        deg0.reshape(N, 1), deg1.reshape(N, 1), x)
    aggs = _agg_kernel(xs0, xs1, xs2, xs3, srcC, dstC, zrows)
    out = _head_call(x, aggs, dis,
                     batch.astype(jnp.int32).reshape(N, 1),
                     W1, b1.reshape(1, D_IN), W2, b2.reshape(1, 1))
    return out


# speedup vs baseline: 14.1461x; 1.3757x over previous
import functools
import jax
import jax.numpy as jnp
from jax import lax
from jax.experimental import pallas as pl
from jax.experimental.pallas import tpu as pltpu
from jax.experimental.pallas import tpu_sc as plsc

N = 10000
E = 160000
D_IN = 256
Q = 64
G = 64
BN = 1000

NC = 2
NS = 16

A_CHUNK = 125
A_ITERS = E // (NC * NS * A_CHUNK)

C_CHUNK = 125
C_ITERS = E // (NS * C_CHUNK)

W_TILES = 10
W_ROWS = N // W_TILES
WCHUNK = 200
WITERS = W_ROWS // WCHUNK

_mesh = plsc.VectorSubcoreMesh(core_axis_name="c", subcore_axis_name="s")


@functools.partial(
    pl.kernel,
    mesh=_mesh,
    out_type=[
        jax.ShapeDtypeStruct((N,), jnp.float32),
        jax.ShapeDtypeStruct((N,), jnp.float32),
    ],
    scratch_types=[
        pltpu.VMEM((A_ITERS, A_CHUNK), jnp.int32),
        pltpu.VMEM((A_CHUNK,), jnp.float32),
        pltpu.VMEM((N,), jnp.float32),
        pltpu.VMEM_SHARED((N,), jnp.float32),
    ],
)
def _deg_kernel(dstA_hbm, ones_hbm, zeros_hbm, deg0_hbm, deg1_hbm,
                idx_v, ones_v, stage_v, deg_sp):
    c = lax.axis_index("c")
    s = lax.axis_index("s")

    @pl.when(s == 0)
    def _():
        pltpu.sync_copy(zeros_hbm, deg_sp)
    plsc.subcore_barrier()

    w = c * NS + s
    pltpu.sync_copy(dstA_hbm.at[w], idx_v)
    pltpu.sync_copy(ones_hbm, ones_v)

    def step(i, _):
        pltpu.sync_copy(ones_v, deg_sp.at[idx_v.at[i]], add=True)
        return 0
    lax.fori_loop(0, A_ITERS, step, 0)

    plsc.subcore_barrier()

    @pl.when((s == 0) & (c == 0))
    def _():
        pltpu.sync_copy(deg_sp, stage_v)
        pltpu.sync_copy(stage_v, deg0_hbm)

    @pl.when((s == 0) & (c == 1))
    def _():
        pltpu.sync_copy(deg_sp, stage_v)
        pltpu.sync_copy(stage_v, deg1_hbm)


@functools.partial(
    pl.kernel,
    mesh=_mesh,
    out_type=[jax.ShapeDtypeStruct((N, Q), jnp.float32) for _ in range(4)],
    scratch_types=[
        pltpu.VMEM((C_ITERS, C_CHUNK), jnp.int32),
        pltpu.VMEM((C_ITERS, C_CHUNK), jnp.int32),
        pltpu.VMEM((C_CHUNK, Q), jnp.float32),
        pltpu.VMEM((C_CHUNK, Q), jnp.float32),
        pltpu.VMEM((WCHUNK, Q), jnp.float32),
        pltpu.SemaphoreType.DMA,
        pltpu.SemaphoreType.DMA,
        pltpu.VMEM_SHARED((N, Q), jnp.float32),
    ],
    compiler_params=pltpu.CompilerParams(use_tc_tiling_on_sc=False),
)
def _agg_kernel(xs0_hbm, xs1_hbm, xs2_hbm, xs3_hbm, srcC_hbm, dstC_hbm,
                zrows_hbm,
                agg0_hbm, agg1_hbm, agg2_hbm, agg3_hbm,
                sidx_v, didx_v, rows0_v, rows1_v, stage_v, sem0, sem1,
                agg_sp):
    c = lax.axis_index("c")
    s = lax.axis_index("s")

    pltpu.sync_copy(srcC_hbm.at[s], sidx_v)
    pltpu.sync_copy(dstC_hbm.at[s], didx_v)

    def one_pass(xsq_hbm, aggq_hbm):
        @pl.when(s < W_TILES)
        def _():
            def zstep(j, _):
                pltpu.sync_copy(
                    zrows_hbm,
                    agg_sp.at[pl.ds(s * W_ROWS + j * WCHUNK, WCHUNK)])
                return 0
            lax.fori_loop(0, WITERS, zstep, 0)
        plsc.subcore_barrier()

        def gather_start(i, buf, sem):
            pltpu.async_copy(xsq_hbm.at[sidx_v.at[i]], buf, sem)

        def gather_wait(i, buf, sem):
            pltpu.make_async_copy(xsq_hbm.at[sidx_v.at[i]], buf, sem).wait()

        gather_start(0, rows0_v, sem0)

        def pair(k, _):
            i0 = 2 * k
            gather_wait(i0, rows0_v, sem0)
            gather_start(i0 + 1, rows1_v, sem1)
            pltpu.sync_copy(rows0_v, agg_sp.at[didx_v.at[i0]], add=True)
            gather_wait(i0 + 1, rows1_v, sem1)

            @pl.when(k < C_ITERS // 2 - 1)
            def _():
                gather_start(i0 + 2, rows0_v, sem0)
            pltpu.sync_copy(rows1_v, agg_sp.at[didx_v.at[i0 + 1]], add=True)
            return 0
        lax.fori_loop(0, C_ITERS // 2, pair, 0)
        plsc.subcore_barrier()

        @pl.when(s < W_TILES)
        def _():
            def wstep(j, _):
                r = s * W_ROWS + j * WCHUNK
                pltpu.sync_copy(agg_sp.at[pl.ds(r, WCHUNK)], stage_v)
                pltpu.sync_copy(stage_v, aggq_hbm.at[pl.ds(r, WCHUNK)])
                return 0
            lax.fori_loop(0, WITERS, wstep, 0)
        plsc.subcore_barrier()

    @pl.when(c == 0)
    def _():
        one_pass(xs0_hbm, agg0_hbm)
        one_pass(xs1_hbm, agg1_hbm)

    @pl.when(c == 1)
    def _():
        one_pass(xs2_hbm, agg2_hbm)
        one_pass(xs3_hbm, agg3_hbm)


def _scale_body(deg0_ref, deg1_ref, x_ref,
                xs0_ref, xs1_ref, xs2_ref, xs3_ref, dis_ref):
    deg = deg0_ref[...] + deg1_ref[...] + 1.0
    dis = lax.rsqrt(deg)
    xv = x_ref[...]
    xs0_ref[...] = dis * xv[:, 0 * Q:1 * Q]
    xs1_ref[...] = dis * xv[:, 1 * Q:2 * Q]
    xs2_ref[...] = dis * xv[:, 2 * Q:3 * Q]
    xs3_ref[...] = dis * xv[:, 3 * Q:4 * Q]
    dis_ref[...] = dis


def _scale_call(deg0, deg1, x):
    return pl.pallas_call(
        _scale_body,
        grid=(N // BN,),
        in_specs=[
            pl.BlockSpec((BN, 1), lambda i: (i, 0)),
            pl.BlockSpec((BN, 1), lambda i: (i, 0)),
            pl.BlockSpec((BN, D_IN), lambda i: (i, 0)),
        ],
        out_specs=[pl.BlockSpec((BN, Q), lambda i: (i, 0))] * 4
        + [pl.BlockSpec((BN, 1), lambda i: (i, 0))],
        out_shape=[jax.ShapeDtypeStruct((N, Q), jnp.float32)] * 4
        + [jax.ShapeDtypeStruct((N, 1), jnp.float32)],
    )(deg0, deg1, x)


def _head_body(x_ref, agg0_ref, agg1_ref, agg2_ref, agg3_ref, dis_ref,
               batch_ref, W1_ref, b1_ref, W2_ref, b2_ref, out_ref, acc_ref):
    i = pl.program_id(0)
    dis = dis_ref[...]
    agg = jnp.concatenate(
        [agg0_ref[...], agg1_ref[...], agg2_ref[...], agg3_ref[...]], axis=1)
    z = agg * dis + (dis * dis) * x_ref[...]
    h = jnp.dot(z, W1_ref[...], preferred_element_type=jnp.float32) \
        + b1_ref[...]
    h = jnp.where(h > 0, h, 0.01 * h)
    v = jnp.dot(h, W2_ref[...], preferred_element_type=jnp.float32)
    b = batch_ref[...]
    gids = lax.broadcasted_iota(jnp.int32, (BN, G), 1)
    onehot = (b == gids).astype(jnp.float32)
    vv = jnp.concatenate([v, jnp.ones((BN, 1), jnp.float32)], axis=1)
    contrib = lax.dot_general(onehot, vv, (((0,), (0,)), ((), ())),
                              preferred_element_type=jnp.float32)

    @pl.when(i == 0)
    def _():
        acc_ref[...] = jnp.zeros_like(acc_ref)

    acc_ref[...] += contrib

    @pl.when(i == pl.num_programs(0) - 1)
    def _():
        a = acc_ref[...]
        out_ref[...] = a[:, 0:1] / jnp.maximum(a[:, 1:2], 1.0) + b2_ref[...]


def _head_call(x, aggs, dis, batch2d, W1, b1r, W2, b2r):
    return pl.pallas_call(
        _head_body,
        grid=(N // BN,),
        in_specs=[
            pl.BlockSpec((BN, D_IN), lambda i: (i, 0)),
            pl.BlockSpec((BN, Q), lambda i: (i, 0)),
            pl.BlockSpec((BN, Q), lambda i: (i, 0)),
            pl.BlockSpec((BN, Q), lambda i: (i, 0)),
            pl.BlockSpec((BN, Q), lambda i: (i, 0)),
            pl.BlockSpec((BN, 1), lambda i: (i, 0)),
            pl.BlockSpec((BN, 1), lambda i: (i, 0)),
            pl.BlockSpec((D_IN, D_IN), lambda i: (0, 0)),
            pl.BlockSpec((1, D_IN), lambda i: (0, 0)),
            pl.BlockSpec((D_IN, 1), lambda i: (0, 0)),
            pl.BlockSpec((1, 1), lambda i: (0, 0)),
        ],
        out_specs=pl.BlockSpec((G, 1), lambda i: (0, 0)),
        out_shape=jax.ShapeDtypeStruct((G, 1), jnp.float32),
        scratch_shapes=[pltpu.VMEM((G, 2), jnp.float32)],
    )(x, *aggs, dis, batch2d, W1, b1r, W2, b2r)


def kernel(x, edge_index, batch, W1, b1, W2, b2):
    ei = edge_index.astype(jnp.int32)
    dstA = ei[1].reshape(NC * NS, A_ITERS, A_CHUNK)
    srcC = ei[0].reshape(NS, C_ITERS, C_CHUNK)
    dstC = ei[1].reshape(NS, C_ITERS, C_CHUNK)
    onesA = jnp.ones((A_CHUNK,), jnp.float32)
    zeros_n = jnp.zeros((N,), jnp.float32)
    zrows = jnp.zeros((WCHUNK, Q), jnp.float32)

    deg0, deg1 = _deg_kernel(dstA, onesA, zeros_n)
    xs0, xs1, xs2, xs3, dis = _scale_call(
        deg0.reshape(N, 1), deg1.reshape(N, 1), x)
    aggs = _agg_kernel(xs0, xs1, xs2, xs3, srcC, dstC, zrows)
    out = _head_call(x, aggs, dis,
                     batch.astype(jnp.int32).reshape(N, 1),
                     W1, b1.reshape(1, D_IN), W2, b2.reshape(1, 1))
    return out


# speedup vs baseline: 20.8952x; 1.4771x over previous
import functools
import jax
import jax.numpy as jnp
from jax import lax
from jax.experimental import pallas as pl
from jax.experimental.pallas import tpu as pltpu
from jax.experimental.pallas import tpu_sc as plsc

N = 10000
E = 160000
D_IN = 256
H = 128
G = 64
BN = 1000

NC = 2
NS = 16

A_CHUNK = 125
A_ITERS = E // (NC * NS * A_CHUNK)

C_CHUNK = 125
C_ITERS = E // (NS * C_CHUNK)

W_TILES = 10
W_ROWS = N // W_TILES
WCHUNK = 200
WITERS = W_ROWS // WCHUNK

_mesh = plsc.VectorSubcoreMesh(core_axis_name="c", subcore_axis_name="s")


@functools.partial(
    pl.kernel,
    mesh=_mesh,
    out_type=[
        jax.ShapeDtypeStruct((N,), jnp.float32),
        jax.ShapeDtypeStruct((N,), jnp.float32),
    ],
    scratch_types=[
        pltpu.VMEM((A_ITERS, A_CHUNK), jnp.int32),
        pltpu.VMEM((A_CHUNK,), jnp.float32),
        pltpu.VMEM((N,), jnp.float32),
        pltpu.VMEM_SHARED((N,), jnp.float32),
    ],
)
def _deg_kernel(dstA_hbm, ones_hbm, zeros_hbm, deg0_hbm, deg1_hbm,
                idx_v, ones_v, stage_v, deg_sp):
    c = lax.axis_index("c")
    s = lax.axis_index("s")

    @pl.when(s == 0)
    def _():
        pltpu.sync_copy(zeros_hbm, deg_sp)
    plsc.subcore_barrier()

    w = c * NS + s
    pltpu.sync_copy(dstA_hbm.at[w], idx_v)
    pltpu.sync_copy(ones_hbm, ones_v)

    def step(i, _):
        pltpu.sync_copy(ones_v, deg_sp.at[idx_v.at[i]], add=True)
        return 0
    lax.fori_loop(0, A_ITERS, step, 0)

    plsc.subcore_barrier()

    @pl.when((s == 0) & (c == 0))
    def _():
        pltpu.sync_copy(deg_sp, stage_v)
        pltpu.sync_copy(stage_v, deg0_hbm)

    @pl.when((s == 0) & (c == 1))
    def _():
        pltpu.sync_copy(deg_sp, stage_v)
        pltpu.sync_copy(stage_v, deg1_hbm)


@functools.partial(
    pl.kernel,
    mesh=_mesh,
    out_type=[
        jax.ShapeDtypeStruct((N, H), jnp.bfloat16),
        jax.ShapeDtypeStruct((N, H), jnp.bfloat16),
    ],
    scratch_types=[
        pltpu.VMEM((C_ITERS, C_CHUNK), jnp.int32),
        pltpu.VMEM((C_ITERS, C_CHUNK), jnp.int32),
        pltpu.VMEM((C_CHUNK, H), jnp.bfloat16),
        pltpu.VMEM((C_CHUNK, H), jnp.bfloat16),
        pltpu.VMEM((WCHUNK, H), jnp.bfloat16),
        pltpu.SemaphoreType.DMA,
        pltpu.SemaphoreType.DMA,
        pltpu.VMEM_SHARED((N, H), jnp.bfloat16),
    ],
    compiler_params=pltpu.CompilerParams(use_tc_tiling_on_sc=False),
)
def _agg_kernel(xs0_hbm, xs1_hbm, srcC_hbm, dstC_hbm, zrows_hbm,
                agg0_hbm, agg1_hbm,
                sidx_v, didx_v, rows0_v, rows1_v, stage_v, sem0, sem1,
                agg_sp):
    c = lax.axis_index("c")
    s = lax.axis_index("s")

    pltpu.sync_copy(srcC_hbm.at[s], sidx_v)
    pltpu.sync_copy(dstC_hbm.at[s], didx_v)

    @pl.when(s < W_TILES)
    def _():
        def zstep(j, _):
            pltpu.sync_copy(
                zrows_hbm,
                agg_sp.at[pl.ds(s * W_ROWS + j * WCHUNK, WCHUNK)])
            return 0
        lax.fori_loop(0, WITERS, zstep, 0)
    plsc.subcore_barrier()

    def one_pass(xsq_hbm, aggq_hbm):
        def gather_start(i, buf, sem):
            pltpu.async_copy(xsq_hbm.at[sidx_v.at[i]], buf, sem)

        def gather_wait(i, buf, sem):
            pltpu.make_async_copy(xsq_hbm.at[sidx_v.at[i]], buf, sem).wait()

        gather_start(0, rows0_v, sem0)

        def pair(k, _):
            i0 = 2 * k
            gather_wait(i0, rows0_v, sem0)
            gather_start(i0 + 1, rows1_v, sem1)
            pltpu.sync_copy(rows0_v, agg_sp.at[didx_v.at[i0]], add=True)
            gather_wait(i0 + 1, rows1_v, sem1)

            @pl.when(k < C_ITERS // 2 - 1)
            def _():
                gather_start(i0 + 2, rows0_v, sem0)
            pltpu.sync_copy(rows1_v, agg_sp.at[didx_v.at[i0 + 1]], add=True)
            return 0
        lax.fori_loop(0, C_ITERS // 2, pair, 0)
        plsc.subcore_barrier()

        @pl.when(s < W_TILES)
        def _():
            def wstep(j, _):
                r = s * W_ROWS + j * WCHUNK
                pltpu.sync_copy(agg_sp.at[pl.ds(r, WCHUNK)], stage_v)
                pltpu.sync_copy(stage_v, aggq_hbm.at[pl.ds(r, WCHUNK)])
                return 0
            lax.fori_loop(0, WITERS, wstep, 0)

    @pl.when(c == 0)
    def _():
        one_pass(xs0_hbm, agg0_hbm)

    @pl.when(c == 1)
    def _():
        one_pass(xs1_hbm, agg1_hbm)


def _scale_body(deg0_ref, deg1_ref, x_ref, xs0_ref, xs1_ref, dis_ref):
    deg = deg0_ref[...] + deg1_ref[...] + 1.0
    dis = lax.rsqrt(deg)
    xv = x_ref[...]
    xs0_ref[...] = (dis * xv[:, :H]).astype(jnp.bfloat16)
    xs1_ref[...] = (dis * xv[:, H:]).astype(jnp.bfloat16)
    dis_ref[...] = dis


def _scale_call(deg0, deg1, x):
    return pl.pallas_call(
        _scale_body,
        grid=(N // BN,),
        in_specs=[
            pl.BlockSpec((BN, 1), lambda i: (i, 0)),
            pl.BlockSpec((BN, 1), lambda i: (i, 0)),
            pl.BlockSpec((BN, D_IN), lambda i: (i, 0)),
        ],
        out_specs=[
            pl.BlockSpec((BN, H), lambda i: (i, 0)),
            pl.BlockSpec((BN, H), lambda i: (i, 0)),
            pl.BlockSpec((BN, 1), lambda i: (i, 0)),
        ],
        out_shape=[
            jax.ShapeDtypeStruct((N, H), jnp.bfloat16),
            jax.ShapeDtypeStruct((N, H), jnp.bfloat16),
            jax.ShapeDtypeStruct((N, 1), jnp.float32),
        ],
    )(deg0, deg1, x)


def _head_body(x_ref, agg0_ref, agg1_ref, dis_ref, batch_ref,
               W1_ref, b1_ref, W2_ref, b2_ref, out_ref, acc_ref):
    i = pl.program_id(0)
    dis = dis_ref[...]
    agg = jnp.concatenate(
        [agg0_ref[...].astype(jnp.float32), agg1_ref[...].astype(jnp.float32)],
        axis=1)
    z = agg * dis + (dis * dis) * x_ref[...]
    h = jnp.dot(z, W1_ref[...], preferred_element_type=jnp.float32) \
        + b1_ref[...]
    h = jnp.where(h > 0, h, 0.01 * h)
    v = jnp.dot(h, W2_ref[...], preferred_element_type=jnp.float32)
    b = batch_ref[...]
    gids = lax.broadcasted_iota(jnp.int32, (BN, G), 1)
    onehot = (b == gids).astype(jnp.float32)
    vv = jnp.concatenate([v, jnp.ones((BN, 1), jnp.float32)], axis=1)
    contrib = lax.dot_general(onehot, vv, (((0,), (0,)), ((), ())),
                              preferred_element_type=jnp.float32)

    @pl.when(i == 0)
    def _():
        acc_ref[...] = jnp.zeros_like(acc_ref)

    acc_ref[...] += contrib

    @pl.when(i == pl.num_programs(0) - 1)
    def _():
        a = acc_ref[...]
        out_ref[...] = a[:, 0:1] / jnp.maximum(a[:, 1:2], 1.0) + b2_ref[...]


def _head_call(x, agg0, agg1, dis, batch2d, W1, b1r, W2, b2r):
    return pl.pallas_call(
        _head_body,
        grid=(N // BN,),
        in_specs=[
            pl.BlockSpec((BN, D_IN), lambda i: (i, 0)),
            pl.BlockSpec((BN, H), lambda i: (i, 0)),
            pl.BlockSpec((BN, H), lambda i: (i, 0)),
            pl.BlockSpec((BN, 1), lambda i: (i, 0)),
            pl.BlockSpec((BN, 1), lambda i: (i, 0)),
            pl.BlockSpec((D_IN, D_IN), lambda i: (0, 0)),
            pl.BlockSpec((1, D_IN), lambda i: (0, 0)),
            pl.BlockSpec((D_IN, 1), lambda i: (0, 0)),
            pl.BlockSpec((1, 1), lambda i: (0, 0)),
        ],
        out_specs=pl.BlockSpec((G, 1), lambda i: (0, 0)),
        out_shape=jax.ShapeDtypeStruct((G, 1), jnp.float32),
        scratch_shapes=[pltpu.VMEM((G, 2), jnp.float32)],
    )(x, agg0, agg1, dis, batch2d, W1, b1r, W2, b2r)


def kernel(x, edge_index, batch, W1, b1, W2, b2):
    ei = edge_index.astype(jnp.int32)
    dstA = ei[1].reshape(NC * NS, A_ITERS, A_CHUNK)
    srcC = ei[0].reshape(NS, C_ITERS, C_CHUNK)
    dstC = ei[1].reshape(NS, C_ITERS, C_CHUNK)
    onesA = jnp.ones((A_CHUNK,), jnp.float32)
    zeros_n = jnp.zeros((N,), jnp.float32)
    zrows = jnp.zeros((WCHUNK, H), jnp.bfloat16)

    deg0, deg1 = _deg_kernel(dstA, onesA, zeros_n)
    xs0, xs1, dis = _scale_call(deg0.reshape(N, 1), deg1.reshape(N, 1), x)
    agg0, agg1 = _agg_kernel(xs0, xs1, srcC, dstC, zrows)
    out = _head_call(x, agg0, agg1, dis,
                     batch.astype(jnp.int32).reshape(N, 1),
                     W1, b1.reshape(1, D_IN), W2, b2.reshape(1, 1))
    return out


# speedup vs baseline: 25.6901x; 1.2295x over previous
import functools
import jax
import jax.numpy as jnp
from jax import lax
from jax.experimental import pallas as pl
from jax.experimental.pallas import tpu as pltpu
from jax.experimental.pallas import tpu_sc as plsc

N = 10000
E = 160000
D_IN = 256
H = 128
G = 64
BN = 2000

NC = 2
NS = 16

A_CHUNK = 125
A_ITERS = E // (NC * NS * A_CHUNK)

C_CHUNK = 125
C_ITERS = E // (NS * C_CHUNK)

W_TILES = 10
W_ROWS = N // W_TILES
WCHUNK = 200
WITERS = W_ROWS // WCHUNK

_mesh = plsc.VectorSubcoreMesh(core_axis_name="c", subcore_axis_name="s")


@functools.partial(
    pl.kernel,
    mesh=_mesh,
    out_type=[
        jax.ShapeDtypeStruct((N,), jnp.float32),
        jax.ShapeDtypeStruct((N,), jnp.float32),
    ],
    scratch_types=[
        pltpu.VMEM((A_ITERS, A_CHUNK), jnp.int32),
        pltpu.VMEM((A_CHUNK,), jnp.float32),
        pltpu.VMEM((N,), jnp.float32),
        pltpu.VMEM_SHARED((N,), jnp.float32),
    ],
)
def _deg_kernel(dstA_hbm, ones_hbm, zeros_hbm, deg0_hbm, deg1_hbm,
                idx_v, ones_v, stage_v, deg_sp):
    c = lax.axis_index("c")
    s = lax.axis_index("s")

    @pl.when(s == 0)
    def _():
        pltpu.sync_copy(zeros_hbm, deg_sp)
    plsc.subcore_barrier()

    w = c * NS + s
    pltpu.sync_copy(dstA_hbm.at[w], idx_v)
    pltpu.sync_copy(ones_hbm, ones_v)

    def step(i, _):
        pltpu.sync_copy(ones_v, deg_sp.at[idx_v.at[i]], add=True)
        return 0
    lax.fori_loop(0, A_ITERS, step, 0)

    plsc.subcore_barrier()

    @pl.when((s == 0) & (c == 0))
    def _():
        pltpu.sync_copy(deg_sp, stage_v)
        pltpu.sync_copy(stage_v, deg0_hbm)

    @pl.when((s == 0) & (c == 1))
    def _():
        pltpu.sync_copy(deg_sp, stage_v)
        pltpu.sync_copy(stage_v, deg1_hbm)


@functools.partial(
    pl.kernel,
    mesh=_mesh,
    out_type=[
        jax.ShapeDtypeStruct((N, H), jnp.bfloat16),
        jax.ShapeDtypeStruct((N, H), jnp.bfloat16),
    ],
    scratch_types=[
        pltpu.VMEM((C_ITERS, C_CHUNK), jnp.int32),
        pltpu.VMEM((C_ITERS, C_CHUNK), jnp.int32),
        pltpu.VMEM((C_CHUNK, H), jnp.bfloat16),
        pltpu.VMEM((C_CHUNK, H), jnp.bfloat16),
        pltpu.VMEM((C_CHUNK, H), jnp.bfloat16),
        pltpu.VMEM((C_CHUNK, H), jnp.bfloat16),
        pltpu.VMEM((WCHUNK, H), jnp.bfloat16),
        pltpu.SemaphoreType.DMA,
        pltpu.SemaphoreType.DMA,
        pltpu.SemaphoreType.DMA,
        pltpu.SemaphoreType.DMA,
        pltpu.VMEM_SHARED((N, H), jnp.bfloat16),
    ],
    compiler_params=pltpu.CompilerParams(use_tc_tiling_on_sc=False),
)
def _agg_kernel(xs0_hbm, xs1_hbm, srcC_hbm, dstC_hbm, zrows_hbm,
                agg0_hbm, agg1_hbm,
                sidx_v, didx_v, rows0_v, rows1_v, rows2_v, rows3_v,
                stage_v, sem0, sem1, sem2, sem3,
                agg_sp):
    c = lax.axis_index("c")
    s = lax.axis_index("s")

    pltpu.sync_copy(srcC_hbm.at[s], sidx_v)
    pltpu.sync_copy(dstC_hbm.at[s], didx_v)

    @pl.when(s < W_TILES)
    def _():
        def zstep(j, _):
            pltpu.sync_copy(
                zrows_hbm,
                agg_sp.at[pl.ds(s * W_ROWS + j * WCHUNK, WCHUNK)])
            return 0
        lax.fori_loop(0, WITERS, zstep, 0)
    plsc.subcore_barrier()

    def one_pass(xsq_hbm, aggq_hbm):
        bufs = (rows0_v, rows1_v, rows2_v, rows3_v)
        sems = (sem0, sem1, sem2, sem3)
        NB = 4
        NK = C_ITERS // NB

        def gather_start(i, b):
            pltpu.async_copy(xsq_hbm.at[sidx_v.at[i]], bufs[b], sems[b])

        def gather_wait(i, b):
            pltpu.make_async_copy(
                xsq_hbm.at[sidx_v.at[i]], bufs[b], sems[b]).wait()

        gather_start(0, 0)
        gather_start(1, 1)

        def quad(k, _):
            i0 = NB * k
            for p in range(NB):
                i = i0 + p
                gather_wait(i, p)
                if p < NB - 2:
                    gather_start(i + 2, p + 2)
                else:
                    @pl.when(k < NK - 1)
                    def _():
                        gather_start(i + 2, p + 2 - NB)
                pltpu.sync_copy(bufs[p], agg_sp.at[didx_v.at[i]], add=True)
            return 0
        lax.fori_loop(0, NK, quad, 0)
        plsc.subcore_barrier()

        @pl.when(s < W_TILES)
        def _():
            def wstep(j, _):
                r = s * W_ROWS + j * WCHUNK
                pltpu.sync_copy(agg_sp.at[pl.ds(r, WCHUNK)], stage_v)
                pltpu.sync_copy(stage_v, aggq_hbm.at[pl.ds(r, WCHUNK)])
                return 0
            lax.fori_loop(0, WITERS, wstep, 0)

    @pl.when(c == 0)
    def _():
        one_pass(xs0_hbm, agg0_hbm)

    @pl.when(c == 1)
    def _():
        one_pass(xs1_hbm, agg1_hbm)


def _scale_body(deg0_ref, deg1_ref, x_ref, xs0_ref, xs1_ref, dis_ref):
    deg = deg0_ref[...] + deg1_ref[...] + 1.0
    dis = lax.rsqrt(deg)
    xv = x_ref[...]
    xs0_ref[...] = (dis * xv[:, :H]).astype(jnp.bfloat16)
    xs1_ref[...] = (dis * xv[:, H:]).astype(jnp.bfloat16)
    dis_ref[...] = dis


def _scale_call(deg0, deg1, x):
    return pl.pallas_call(
        _scale_body,
        grid=(N // BN,),
        in_specs=[
            pl.BlockSpec((BN, 1), lambda i: (i, 0)),
            pl.BlockSpec((BN, 1), lambda i: (i, 0)),
            pl.BlockSpec((BN, D_IN), lambda i: (i, 0)),
        ],
        out_specs=[
            pl.BlockSpec((BN, H), lambda i: (i, 0)),
            pl.BlockSpec((BN, H), lambda i: (i, 0)),
            pl.BlockSpec((BN, 1), lambda i: (i, 0)),
        ],
        out_shape=[
            jax.ShapeDtypeStruct((N, H), jnp.bfloat16),
            jax.ShapeDtypeStruct((N, H), jnp.bfloat16),
            jax.ShapeDtypeStruct((N, 1), jnp.float32),
        ],
    )(deg0, deg1, x)


def _head_body(xs0_ref, xs1_ref, agg0_ref, agg1_ref, dis_ref, batch_ref,
               W1_ref, b1_ref, W2_ref, b2_ref, out_ref, acc_ref):
    i = pl.program_id(0)
    dis = dis_ref[...]
    agg = jnp.concatenate(
        [agg0_ref[...].astype(jnp.float32) + xs0_ref[...].astype(jnp.float32),
         agg1_ref[...].astype(jnp.float32) + xs1_ref[...].astype(jnp.float32)],
        axis=1)
    z = agg * dis
    h = jnp.dot(z, W1_ref[...], preferred_element_type=jnp.float32) \
        + b1_ref[...]
    h = jnp.where(h > 0, h, 0.01 * h)
    v = jnp.dot(h, W2_ref[...], preferred_element_type=jnp.float32)
    b = batch_ref[...]
    gids = lax.broadcasted_iota(jnp.int32, (BN, G), 1)
    onehot = (b == gids).astype(jnp.float32)
    vv = jnp.concatenate([v, jnp.ones((BN, 1), jnp.float32)], axis=1)
    contrib = lax.dot_general(onehot, vv, (((0,), (0,)), ((), ())),
                              preferred_element_type=jnp.float32)

    @pl.when(i == 0)
    def _():
        acc_ref[...] = jnp.zeros_like(acc_ref)

    acc_ref[...] += contrib

    @pl.when(i == pl.num_programs(0) - 1)
    def _():
        a = acc_ref[...]
        out_ref[...] = a[:, 0:1] / jnp.maximum(a[:, 1:2], 1.0) + b2_ref[...]


def _head_call(xs0, xs1, agg0, agg1, dis, batch2d, W1, b1r, W2, b2r):
    return pl.pallas_call(
        _head_body,
        grid=(N // BN,),
        in_specs=[
            pl.BlockSpec((BN, H), lambda i: (i, 0)),
            pl.BlockSpec((BN, H), lambda i: (i, 0)),
            pl.BlockSpec((BN, H), lambda i: (i, 0)),
            pl.BlockSpec((BN, H), lambda i: (i, 0)),
            pl.BlockSpec((BN, 1), lambda i: (i, 0)),
            pl.BlockSpec((BN, 1), lambda i: (i, 0)),
            pl.BlockSpec((D_IN, D_IN), lambda i: (0, 0)),
            pl.BlockSpec((1, D_IN), lambda i: (0, 0)),
            pl.BlockSpec((D_IN, 1), lambda i: (0, 0)),
            pl.BlockSpec((1, 1), lambda i: (0, 0)),
        ],
        out_specs=pl.BlockSpec((G, 1), lambda i: (0, 0)),
        out_shape=jax.ShapeDtypeStruct((G, 1), jnp.float32),
        scratch_shapes=[pltpu.VMEM((G, 2), jnp.float32)],
    )(xs0, xs1, agg0, agg1, dis, batch2d, W1, b1r, W2, b2r)


def kernel(x, edge_index, batch, W1, b1, W2, b2):
    ei = edge_index.astype(jnp.int32)
    dstA = ei[1].reshape(NC * NS, A_ITERS, A_CHUNK)
    srcC = ei[0].reshape(NS, C_ITERS, C_CHUNK)
    dstC = ei[1].reshape(NS, C_ITERS, C_CHUNK)
    onesA = jnp.ones((A_CHUNK,), jnp.float32)
    zeros_n = jnp.zeros((N,), jnp.float32)
    zrows = jnp.zeros((WCHUNK, H), jnp.bfloat16)

    deg0, deg1 = _deg_kernel(dstA, onesA, zeros_n)
    xs0, xs1, dis = _scale_call(deg0.reshape(N, 1), deg1.reshape(N, 1), x)
    agg0, agg1 = _agg_kernel(xs0, xs1, srcC, dstC, zrows)
    out = _head_call(xs0, xs1, agg0, agg1, dis,
                     batch.astype(jnp.int32).reshape(N, 1),
                     W1, b1.reshape(1, D_IN), W2, b2.reshape(1, 1))
    return out
